# baseline (speedup 1.0000x reference)
import functools

import jax
import jax.numpy as jnp
from jax import lax
from jax.experimental import pallas as pl
from jax.experimental.pallas import tpu as pltpu
from jax.experimental.pallas import tpu_sc as plsc

_MINT = -2147483648


def _keys_body(conf_ref, fcw_ref, fcb_ref, lam_ref, keys_ref, *, N, B, H, W):
    Vb = N // B
    conf = conf_ref[...]
    cm = jax.nn.sigmoid(conf).max(axis=1)
    lam = lam_ref[0]
    fcw = fcw_ref[...]
    fcb = fcb_ref[...]

    tf = (
        jnp.dot(cm.reshape(N * H, W), fcw.T, preferred_element_type=jnp.float32)
        + fcb[None, :]
    ).reshape(N, H, H)

    comms = []
    for v in range(N):
        ego = (v // Vb) * Vb
        req = 1.0 - cm[ego]
        diff = cm[v] * req
        sim = jnp.dot(tf[v], cm[v], preferred_element_type=jnp.float32)
        comms.append(lam * diff + (1.0 - lam) * sim)
    comm = jnp.stack(comms, axis=0)

    ki = lax.bitcast_convert_type(comm, jnp.int32)
    k2 = ki ^ ((ki >> 31) | jnp.int32(_MINT))
    keys_ref[...] = lax.bitcast_convert_type(k2, jnp.uint32)


def _sc_topk_body(keys_hbm, mask_hbm, keys_v, mask_v, hist_v, *, N, Vb, HW, K):
    L = 16
    n_chunks = HW // L
    wid = lax.axis_index("s") * 2 + lax.axis_index("c")

    @pl.when(wid < N)
    def _():
        row = wid
        pltpu.sync_copy(keys_hbm.at[row], keys_v)
        lane = lax.broadcasted_iota(jnp.int32, (L,), 0)

        def one_pass(shift, carry):
            prefix, himask, k_rem = carry

            def zero_step(i, _):
                hist_v[pl.ds(i * L, L)] = jnp.zeros((L,), jnp.int32)
                return 0

            lax.fori_loop(0, 256, zero_step, 0)

            def hist_step(i, _):
                k = keys_v[pl.ds(i * L, L)]
                match = (k & himask) == prefix
                digit = ((k >> jnp.uint32(shift))
                         & jnp.uint32(255)).astype(jnp.int32)
                addr = lane * 256 + digit
                plsc.addupdate_scatter(hist_v, [addr], jnp.ones((L,), jnp.int32),
                                       mask=match)
                return 0

            lax.fori_loop(0, n_chunks, hist_step, 0)

            def scan_step(j2, c):
                seen, bin_acc, above_acc = c
                j = 15 - j2
                tv = jnp.zeros((L,), jnp.int32)
                for l in range(L):
                    tv = tv + hist_v[pl.ds(l * 256 + j * L, L)]
                rv = lax.rev(tv, (0,))
                incl = seen + plsc.cumsum(rv)
                excl = incl - rv
                hit = (excl < k_rem) & (incl >= k_rem)
                bins_desc = j * L + 15 - lane
                bin_acc = bin_acc + jnp.sum(jnp.where(hit, bins_desc, 0))
                above_acc = above_acc + jnp.sum(jnp.where(hit, excl, 0))
                return (seen + jnp.sum(tv), bin_acc, above_acc)

            _, d, above = lax.fori_loop(
                0, 16, scan_step,
                (jnp.int32(0), jnp.int32(0), jnp.int32(0)))

            prefix = prefix | (d.astype(jnp.uint32) << jnp.uint32(shift))
            himask = himask | (jnp.uint32(255) << jnp.uint32(shift))
            return (prefix, himask, k_rem - above)

        carry = (jnp.uint32(0), jnp.uint32(0), jnp.int32(K))
        for shift in (24, 16, 8, 0):
            carry = one_pass(shift, carry)
        t_key, _, need = carry

        is_ego = (row % Vb) == 0

        def mask_step(i, running):
            k = keys_v[pl.ds(i * L, L)]
            gt = k > t_key
            eq = k == t_key
            eqi = eq.astype(jnp.int32)
            cs = running + plsc.cumsum(eqi)
            sel = gt | (eq & (cs <= need)) | is_ego
            mask_v[pl.ds(i * L, L)] = jnp.where(sel, 1.0, 0.0).astype(jnp.float32)
            return running + jnp.sum(eqi)

        lax.fori_loop(0, n_chunks, mask_step, jnp.int32(0))
        pltpu.sync_copy(mask_v, mask_hbm.at[row])


def _apply_body(x_ref, mask_ref, xm_ref, fuse_ref):
    xm = x_ref[...] * mask_ref[...][:, None, :, :]
    xm_ref[...] = xm
    fuse_ref[...] = jnp.max(xm, axis=0, keepdims=True)


@jax.jit
def kernel(x, record_len, conf_map, lam, fc_w, fc_b):
    N, C, H, W = x.shape
    B = record_len.shape[0]
    Vb = N // B
    HW = H * W
    K = HW // 2

    keys = pl.pallas_call(
        functools.partial(_keys_body, N=N, B=B, H=H, W=W),
        out_shape=jax.ShapeDtypeStruct((N, H, W), jnp.uint32),
        in_specs=[
            pl.BlockSpec(memory_space=pltpu.VMEM),
            pl.BlockSpec(memory_space=pltpu.VMEM),
            pl.BlockSpec(memory_space=pltpu.VMEM),
            pl.BlockSpec(memory_space=pltpu.SMEM),
        ],
        out_specs=pl.BlockSpec(memory_space=pltpu.VMEM),
    )(conf_map, fc_w, fc_b, lam.reshape(1))

    sc_topk = functools.partial(
        pl.kernel,
        mesh=plsc.VectorSubcoreMesh(core_axis_name="c", subcore_axis_name="s"),
        out_type=jax.ShapeDtypeStruct((N, HW), jnp.float32),
        scratch_types=[
            pltpu.VMEM((HW,), jnp.uint32),
            pltpu.VMEM((HW,), jnp.float32),
            pltpu.VMEM((4096,), jnp.int32),
        ],
        compiler_params=pltpu.CompilerParams(needs_layout_passes=False),
    )(functools.partial(_sc_topk_body, N=N, Vb=Vb, HW=HW, K=K))
    mask = sc_topk(keys.reshape(N, HW)).reshape(N, H, W)

    CB = 8
    xm, x_fuse = pl.pallas_call(
        _apply_body,
        grid=(B, C // CB),
        in_specs=[
            pl.BlockSpec((Vb, CB, H, W), lambda b, c: (b, c, 0, 0)),
            pl.BlockSpec((Vb, H, W), lambda b, c: (b, 0, 0)),
        ],
        out_specs=[
            pl.BlockSpec((Vb, CB, H, W), lambda b, c: (b, c, 0, 0)),
            pl.BlockSpec((1, CB, H, W), lambda b, c: (b, c, 0, 0)),
        ],
        out_shape=[
            jax.ShapeDtypeStruct((N, C, H, W), jnp.float32),
            jax.ShapeDtypeStruct((B, C, H, W), jnp.float32),
        ],
        compiler_params=pltpu.CompilerParams(
            dimension_semantics=("parallel", "parallel"),
        ),
    )(x, mask)

    hw = jnp.float32(HW)
    rates = jnp.float32(Vb * K) / (record_len.astype(jnp.float32) * hw)
    communication_rates = jnp.sum(rates) / jnp.float32(B)

    return (x_fuse, communication_rates, xm, jnp.float32(0.0))


# speedup vs baseline: 1.1594x; 1.1594x over previous
import functools

import jax
import jax.numpy as jnp
from jax import lax
from jax.experimental import pallas as pl
from jax.experimental.pallas import tpu as pltpu
from jax.experimental.pallas import tpu_sc as plsc

_MINT = -2147483648


def _keys_body(conf_ref, fcw_ref, fcb_ref, lam_ref, keys_ref, *, N, B, H, W):
    Vb = N // B
    conf = conf_ref[...]
    cm = jax.nn.sigmoid(conf).max(axis=1)
    lam = lam_ref[0]
    fcw = fcw_ref[...]
    fcb = fcb_ref[...]

    tf = (
        jnp.dot(cm.reshape(N * H, W), fcw.T, preferred_element_type=jnp.float32)
        + fcb[None, :]
    ).reshape(N, H, H)

    comms = []
    for v in range(N):
        ego = (v // Vb) * Vb
        req = 1.0 - cm[ego]
        diff = cm[v] * req
        sim = jnp.dot(tf[v], cm[v], preferred_element_type=jnp.float32)
        comms.append(lam * diff + (1.0 - lam) * sim)
    comm = jnp.stack(comms, axis=0)

    ki = lax.bitcast_convert_type(comm, jnp.int32)
    k2 = ki ^ ((ki >> 31) | jnp.int32(_MINT))
    keys_ref[...] = lax.bitcast_convert_type(k2, jnp.uint32)


def _sc_topk_body(keys_hbm, mask_hbm, keys_v, mask_v, hist_v, pref_v,
                  *, N, Vb, HW, K):
    L = 16
    U = 4
    n_chunks = HW // L
    wid = lax.axis_index("s") * 2 + lax.axis_index("c")

    @pl.when(wid < N)
    def _():
        row = wid
        pltpu.sync_copy(keys_hbm.at[row], keys_v)
        lane = lax.broadcasted_iota(jnp.int32, (L,), 0)
        ones = jnp.ones((L,), jnp.int32)

        def one_pass(shift, carry):
            prefix, k_rem = carry
            wmax = jnp.uint32((1 << (shift + 8)) - 1)

            @plsc.parallel_loop(0, U * 256, unroll=8)
            def _(i):
                hist_v[pl.ds(i * L, L)] = jnp.zeros((L,), jnp.int32)

            @plsc.parallel_loop(0, n_chunks, unroll=U)
            def _(i):
                k = keys_v[pl.ds(i * L, L)]
                match = (k - prefix) <= wmax
                digit = ((k >> jnp.uint32(shift))
                         & jnp.uint32(255)).astype(jnp.int32)
                addr = (i & (U - 1)) * (256 * L) + lane * 256 + digit
                plsc.addupdate_scatter(hist_v, [addr], ones, mask=match)

            def scan_step(j2, c):
                seen, bin_acc, above_acc, size_acc = c
                j = 15 - j2
                tv = jnp.zeros((L,), jnp.int32)
                for u in range(U):
                    for l in range(L):
                        tv = tv + hist_v[pl.ds(u * 256 * L + l * 256 + j * L, L)]
                rv = lax.rev(tv, (0,))
                incl = seen + plsc.cumsum(rv)
                excl = incl - rv
                hit = (excl < k_rem) & (incl >= k_rem)
                bins_desc = j * L + 15 - lane
                bin_acc = bin_acc + jnp.sum(jnp.where(hit, bins_desc, 0))
                above_acc = above_acc + jnp.sum(jnp.where(hit, excl, 0))
                size_acc = size_acc + jnp.sum(jnp.where(hit, rv, 0))
                return (seen + jnp.sum(tv), bin_acc, above_acc, size_acc)

            _, d, above, dsize = lax.fori_loop(
                0, 16, scan_step,
                (jnp.int32(0), jnp.int32(0), jnp.int32(0), jnp.int32(0)))

            prefix = prefix | (d.astype(jnp.uint32) << jnp.uint32(shift))
            return (prefix, k_rem - above), dsize

        carry = (jnp.uint32(0), jnp.int32(K))
        for shift in (24, 16, 8, 0):
            carry, eq_total = one_pass(shift, carry)
        t_key, need = carry

        is_ego = (row % Vb) == 0

        @pl.when(eq_total == need)
        def _():
            @plsc.parallel_loop(0, n_chunks, unroll=4)
            def _(i):
                k = keys_v[pl.ds(i * L, L)]
                sel = (k >= t_key) | is_ego
                mask_v[pl.ds(i * L, L)] = jnp.where(sel, 1.0, 0.0)

        @pl.when(eq_total != need)
        def _():
            def count_step(i, run):
                k = keys_v[pl.ds(i * L, L)]
                eq = k == t_key
                pref_v[pl.ds(i * L, L)] = run
                return run + plsc.all_reduce_population_count(eq)

            lax.fori_loop(0, n_chunks, count_step, jnp.zeros((L,), jnp.int32))

            @plsc.parallel_loop(0, n_chunks, unroll=2)
            def _(i):
                k = keys_v[pl.ds(i * L, L)]
                eq = k == t_key
                cs = pref_v[pl.ds(i * L, L)] + plsc.cumsum(eq.astype(jnp.int32))
                sel = (k > t_key) | (eq & (cs <= need)) | is_ego
                mask_v[pl.ds(i * L, L)] = jnp.where(sel, 1.0, 0.0)

        pltpu.sync_copy(mask_v, mask_hbm.at[row])


def _apply_body(x_ref, mask_ref, xm_ref, fuse_ref):
    xm = x_ref[...] * mask_ref[...][:, None, :, :]
    xm_ref[...] = xm
    fuse_ref[...] = jnp.max(xm, axis=0, keepdims=True)


@jax.jit
def kernel(x, record_len, conf_map, lam, fc_w, fc_b):
    N, C, H, W = x.shape
    B = record_len.shape[0]
    Vb = N // B
    HW = H * W
    K = HW // 2

    keys = pl.pallas_call(
        functools.partial(_keys_body, N=N, B=B, H=H, W=W),
        out_shape=jax.ShapeDtypeStruct((N, H, W), jnp.uint32),
        in_specs=[
            pl.BlockSpec(memory_space=pltpu.VMEM),
            pl.BlockSpec(memory_space=pltpu.VMEM),
            pl.BlockSpec(memory_space=pltpu.VMEM),
            pl.BlockSpec(memory_space=pltpu.SMEM),
        ],
        out_specs=pl.BlockSpec(memory_space=pltpu.VMEM),
    )(conf_map, fc_w, fc_b, lam.reshape(1))

    sc_topk = functools.partial(
        pl.kernel,
        mesh=plsc.VectorSubcoreMesh(core_axis_name="c", subcore_axis_name="s"),
        out_type=jax.ShapeDtypeStruct((N, HW), jnp.float32),
        scratch_types=[
            pltpu.VMEM((HW,), jnp.uint32),
            pltpu.VMEM((HW,), jnp.float32),
            pltpu.VMEM((4 * 16 * 256,), jnp.int32),
            pltpu.VMEM((HW,), jnp.int32),
        ],
        compiler_params=pltpu.CompilerParams(needs_layout_passes=False),
    )(functools.partial(_sc_topk_body, N=N, Vb=Vb, HW=HW, K=K))
    mask = sc_topk(keys.reshape(N, HW)).reshape(N, H, W)

    CB = 8
    xm, x_fuse = pl.pallas_call(
        _apply_body,
        grid=(B, C // CB),
        in_specs=[
            pl.BlockSpec((Vb, CB, H, W), lambda b, c: (b, c, 0, 0)),
            pl.BlockSpec((Vb, H, W), lambda b, c: (b, 0, 0)),
        ],
        out_specs=[
            pl.BlockSpec((Vb, CB, H, W), lambda b, c: (b, c, 0, 0)),
            pl.BlockSpec((1, CB, H, W), lambda b, c: (b, c, 0, 0)),
        ],
        out_shape=[
            jax.ShapeDtypeStruct((N, C, H, W), jnp.float32),
            jax.ShapeDtypeStruct((B, C, H, W), jnp.float32),
        ],
        compiler_params=pltpu.CompilerParams(
            dimension_semantics=("parallel", "parallel"),
        ),
    )(x, mask)

    hw = jnp.float32(HW)
    rates = jnp.float32(Vb * K) / (record_len.astype(jnp.float32) * hw)
    communication_rates = jnp.sum(rates) / jnp.float32(B)

    return (x_fuse, communication_rates, xm, jnp.float32(0.0))


# speedup vs baseline: 1.2149x; 1.0478x over previous
import functools

import jax
import jax.numpy as jnp
from jax import lax
from jax.experimental import pallas as pl
from jax.experimental.pallas import tpu as pltpu
from jax.experimental.pallas import tpu_sc as plsc

_MINT = -2147483648


def _keys_body(conf_ref, fcw_ref, fcb_ref, lam_ref, keys_ref, *, N, B, H, W):
    Vb = N // B
    conf = conf_ref[...]
    cm = jax.nn.sigmoid(conf).max(axis=1)
    lam = lam_ref[0]
    fcw = fcw_ref[...]
    fcb = fcb_ref[...]

    tf = (
        jnp.dot(cm.reshape(N * H, W), fcw.T, preferred_element_type=jnp.float32)
        + fcb[None, :]
    ).reshape(N, H, H)

    comms = []
    for v in range(N):
        ego = (v // Vb) * Vb
        req = 1.0 - cm[ego]
        diff = cm[v] * req
        sim = jnp.dot(tf[v], cm[v], preferred_element_type=jnp.float32)
        comms.append(lam * diff + (1.0 - lam) * sim)
    comm = jnp.stack(comms, axis=0)

    ki = lax.bitcast_convert_type(comm, jnp.int32)
    k2 = ki ^ ((ki >> 31) | jnp.int32(_MINT))
    keys_ref[...] = lax.bitcast_convert_type(k2, jnp.uint32)


def _sc_topk_body(keys_hbm, mask_hbm, keys_v, mask_v, hist_v, pref_v,
                  *, N, Vb, HW, K):
    L = 16
    U = 4
    n_chunks = HW // L
    wid = lax.axis_index("s") * 2 + lax.axis_index("c")

    @pl.when(wid < N)
    def _():
        row = wid
        pltpu.sync_copy(keys_hbm.at[row], keys_v)
        lane = lax.broadcasted_iota(jnp.int32, (L,), 0)
        ones = jnp.ones((L,), jnp.int32)

        def one_pass(shift, carry):
            prefix, k_rem = carry
            wmax = jnp.uint32((1 << (shift + 8)) - 1)

            @plsc.parallel_loop(0, U * 256, unroll=8)
            def _(i):
                hist_v[pl.ds(i * L, L)] = jnp.zeros((L,), jnp.int32)

            @plsc.parallel_loop(0, n_chunks, unroll=U)
            def _(i):
                k = keys_v[pl.ds(i * L, L)]
                match = (k - prefix) <= wmax
                digit = ((k >> jnp.uint32(shift))
                         & jnp.uint32(255)).astype(jnp.int32)
                addr = (i & (U - 1)) * (256 * L) + lane * 256 + digit
                plsc.addupdate_scatter(hist_v, [addr], ones, mask=match)

            def scan_step(j2, c):
                seen, bin_acc, above_acc, size_acc = c
                j = 15 - j2
                tv = jnp.zeros((L,), jnp.int32)
                for u in range(U):
                    for l in range(L):
                        tv = tv + hist_v[pl.ds(u * 256 * L + l * 256 + j * L, L)]
                rv = lax.rev(tv, (0,))
                incl = seen + plsc.cumsum(rv)
                excl = incl - rv
                hit = (excl < k_rem) & (incl >= k_rem)
                bins_desc = j * L + 15 - lane
                bin_acc = bin_acc + jnp.sum(jnp.where(hit, bins_desc, 0))
                above_acc = above_acc + jnp.sum(jnp.where(hit, excl, 0))
                size_acc = size_acc + jnp.sum(jnp.where(hit, rv, 0))
                return (seen + jnp.sum(tv), bin_acc, above_acc, size_acc)

            _, d, above, dsize = lax.fori_loop(
                0, 16, scan_step,
                (jnp.int32(0), jnp.int32(0), jnp.int32(0), jnp.int32(0)))

            prefix = prefix | (d.astype(jnp.uint32) << jnp.uint32(shift))
            return (prefix, k_rem - above), dsize

        carry = (jnp.uint32(0), jnp.int32(K))
        for shift in (24, 16, 8, 0):
            carry, eq_total = one_pass(shift, carry)
        t_key, need = carry

        is_ego = (row % Vb) == 0

        @pl.when(eq_total == need)
        def _():
            @plsc.parallel_loop(0, n_chunks, unroll=4)
            def _(i):
                k = keys_v[pl.ds(i * L, L)]
                sel = (k >= t_key) | is_ego
                mask_v[pl.ds(i * L, L)] = jnp.where(sel, 1.0, 0.0)

        @pl.when(eq_total != need)
        def _():
            def count_step(i, run):
                k = keys_v[pl.ds(i * L, L)]
                eq = k == t_key
                pref_v[pl.ds(i * L, L)] = run
                return run + plsc.all_reduce_population_count(eq)

            lax.fori_loop(0, n_chunks, count_step, jnp.zeros((L,), jnp.int32))

            @plsc.parallel_loop(0, n_chunks, unroll=2)
            def _(i):
                k = keys_v[pl.ds(i * L, L)]
                eq = k == t_key
                cs = pref_v[pl.ds(i * L, L)] + plsc.cumsum(eq.astype(jnp.int32))
                sel = (k > t_key) | (eq & (cs <= need)) | is_ego
                mask_v[pl.ds(i * L, L)] = jnp.where(sel, 1.0, 0.0)

        pltpu.sync_copy(mask_v, mask_hbm.at[row])


def _apply_body(x_ref, mask_ref, xm_ref, fuse_ref):
    xm = x_ref[...] * mask_ref[...][:, None, :, :]
    xm_ref[...] = xm
    fuse_ref[...] = jnp.max(xm, axis=0, keepdims=True)


@jax.jit
def kernel(x, record_len, conf_map, lam, fc_w, fc_b):
    N, C, H, W = x.shape
    B = record_len.shape[0]
    Vb = N // B
    HW = H * W
    K = HW // 2

    keys = pl.pallas_call(
        functools.partial(_keys_body, N=N, B=B, H=H, W=W),
        out_shape=jax.ShapeDtypeStruct((N, H, W), jnp.uint32),
        in_specs=[
            pl.BlockSpec(memory_space=pltpu.VMEM),
            pl.BlockSpec(memory_space=pltpu.VMEM),
            pl.BlockSpec(memory_space=pltpu.VMEM),
            pl.BlockSpec(memory_space=pltpu.SMEM),
        ],
        out_specs=pl.BlockSpec(memory_space=pltpu.VMEM),
    )(conf_map, fc_w, fc_b, lam.reshape(1))

    sc_topk = functools.partial(
        pl.kernel,
        mesh=plsc.VectorSubcoreMesh(core_axis_name="c", subcore_axis_name="s"),
        out_type=jax.ShapeDtypeStruct((N, HW), jnp.float32),
        scratch_types=[
            pltpu.VMEM((HW,), jnp.uint32),
            pltpu.VMEM((HW,), jnp.float32),
            pltpu.VMEM((4 * 16 * 256,), jnp.int32),
            pltpu.VMEM((HW,), jnp.int32),
        ],
        compiler_params=pltpu.CompilerParams(needs_layout_passes=False),
    )(functools.partial(_sc_topk_body, N=N, Vb=Vb, HW=HW, K=K))
    mask = sc_topk(keys.reshape(N, HW)).reshape(N, H, W)

    CB = 16
    xm, x_fuse = pl.pallas_call(
        _apply_body,
        grid=(B, C // CB),
        in_specs=[
            pl.BlockSpec((Vb, CB, H, W), lambda b, c: (b, c, 0, 0)),
            pl.BlockSpec((Vb, H, W), lambda b, c: (b, 0, 0)),
        ],
        out_specs=[
            pl.BlockSpec((Vb, CB, H, W), lambda b, c: (b, c, 0, 0)),
            pl.BlockSpec((1, CB, H, W), lambda b, c: (b, c, 0, 0)),
        ],
        out_shape=[
            jax.ShapeDtypeStruct((N, C, H, W), jnp.float32),
            jax.ShapeDtypeStruct((B, C, H, W), jnp.float32),
        ],
        compiler_params=pltpu.CompilerParams(
            dimension_semantics=("parallel", "parallel"),
        ),
    )(x, mask)

    hw = jnp.float32(HW)
    rates = jnp.float32(Vb * K) / (record_len.astype(jnp.float32) * hw)
    communication_rates = jnp.sum(rates) / jnp.float32(B)

    return (x_fuse, communication_rates, xm, jnp.float32(0.0))


# speedup vs baseline: 1.2341x; 1.0158x over previous
import functools

import jax
import jax.numpy as jnp
from jax import lax
from jax.experimental import pallas as pl
from jax.experimental.pallas import tpu as pltpu
from jax.experimental.pallas import tpu_sc as plsc

_MINT = -2147483648


def _keys_body(conf_ref, fcw_ref, fcb_ref, lam_ref, keys_ref, *, N, B, H, W):
    Vb = N // B
    conf = conf_ref[...]
    cm = jax.nn.sigmoid(conf).max(axis=1)
    lam = lam_ref[0]
    fcw = fcw_ref[...]
    fcb = fcb_ref[...]

    tf = (
        jnp.dot(cm.reshape(N * H, W), fcw.T, preferred_element_type=jnp.float32)
        + fcb[None, :]
    ).reshape(N, H, H)

    comms = []
    for v in range(N):
        ego = (v // Vb) * Vb
        req = 1.0 - cm[ego]
        diff = cm[v] * req
        sim = jnp.dot(tf[v], cm[v], preferred_element_type=jnp.float32)
        comms.append(lam * diff + (1.0 - lam) * sim)
    comm = jnp.stack(comms, axis=0)

    ki = lax.bitcast_convert_type(comm, jnp.int32)
    k2 = ki ^ ((ki >> 31) | jnp.int32(_MINT))
    keys_ref[...] = lax.bitcast_convert_type(k2, jnp.uint32)


def _sc_topk_body(keys_hbm, mask_hbm, keys_v, mask_v, hist_v, pref_v,
                  *, N, Vb, HW, K):
    L = 16
    U = 4
    n_chunks = HW // L
    wid = lax.axis_index("s") * 2 + lax.axis_index("c")

    @pl.when(wid < N)
    def _():
        row = wid
        pltpu.sync_copy(keys_hbm.at[row], keys_v)
        lane = lax.broadcasted_iota(jnp.int32, (L,), 0)
        ones = jnp.ones((L,), jnp.int32)

        def one_pass(shift, carry):
            prefix, k_rem = carry
            wmax = jnp.uint32((1 << (shift + 8)) - 1)

            @plsc.parallel_loop(0, U * 256, unroll=8)
            def _(i):
                hist_v[pl.ds(i * L, L)] = jnp.zeros((L,), jnp.int32)

            @plsc.parallel_loop(0, n_chunks, unroll=U)
            def _(i):
                k = keys_v[pl.ds(i * L, L)]
                match = (k - prefix) <= wmax
                digit = ((k >> jnp.uint32(shift))
                         & jnp.uint32(255)).astype(jnp.int32)
                addr = (i & (U - 1)) * (256 * L) + lane * 256 + digit
                plsc.addupdate_scatter(hist_v, [addr], ones, mask=match)

            def scan_step(j2, c):
                seen, bin_acc, above_acc, size_acc = c
                j = 15 - j2
                tv = jnp.zeros((L,), jnp.int32)
                for u in range(U):
                    for l in range(L):
                        tv = tv + hist_v[pl.ds(u * 256 * L + l * 256 + j * L, L)]
                rv = lax.rev(tv, (0,))
                incl = seen + plsc.cumsum(rv)
                excl = incl - rv
                hit = (excl < k_rem) & (incl >= k_rem)
                bins_desc = j * L + 15 - lane
                bin_acc = bin_acc + jnp.sum(jnp.where(hit, bins_desc, 0))
                above_acc = above_acc + jnp.sum(jnp.where(hit, excl, 0))
                size_acc = size_acc + jnp.sum(jnp.where(hit, rv, 0))
                return (seen + jnp.sum(tv), bin_acc, above_acc, size_acc)

            _, d, above, dsize = lax.fori_loop(
                0, 16, scan_step,
                (jnp.int32(0), jnp.int32(0), jnp.int32(0), jnp.int32(0)))

            prefix = prefix | (d.astype(jnp.uint32) << jnp.uint32(shift))
            return (prefix, k_rem - above), dsize

        carry = (jnp.uint32(0), jnp.int32(K))
        for shift in (24, 16, 8, 0):
            carry, eq_total = one_pass(shift, carry)
        t_key, need = carry

        is_ego = (row % Vb) == 0

        @pl.when(eq_total == need)
        def _():
            @plsc.parallel_loop(0, n_chunks, unroll=4)
            def _(i):
                k = keys_v[pl.ds(i * L, L)]
                sel = (k >= t_key) | is_ego
                mask_v[pl.ds(i * L, L)] = jnp.where(sel, 1.0, 0.0)

        @pl.when(eq_total != need)
        def _():
            def count_step(i, run):
                k = keys_v[pl.ds(i * L, L)]
                eq = k == t_key
                pref_v[pl.ds(i * L, L)] = run
                return run + plsc.all_reduce_population_count(eq)

            lax.fori_loop(0, n_chunks, count_step, jnp.zeros((L,), jnp.int32))

            @plsc.parallel_loop(0, n_chunks, unroll=2)
            def _(i):
                k = keys_v[pl.ds(i * L, L)]
                eq = k == t_key
                cs = pref_v[pl.ds(i * L, L)] + plsc.cumsum(eq.astype(jnp.int32))
                sel = (k > t_key) | (eq & (cs <= need)) | is_ego
                mask_v[pl.ds(i * L, L)] = jnp.where(sel, 1.0, 0.0)

        pltpu.sync_copy(mask_v, mask_hbm.at[row])


def _apply_body(x_ref, mask_ref, xm_ref, fuse_ref):
    xm = x_ref[...] * mask_ref[...][:, None, :, :]
    xm_ref[...] = xm
    fuse_ref[...] = jnp.max(xm, axis=0, keepdims=True)


@jax.jit
def kernel(x, record_len, conf_map, lam, fc_w, fc_b):
    N, C, H, W = x.shape
    B = record_len.shape[0]
    Vb = N // B
    HW = H * W
    K = HW // 2

    keys = pl.pallas_call(
        functools.partial(_keys_body, N=N, B=B, H=H, W=W),
        out_shape=jax.ShapeDtypeStruct((N, H, W), jnp.uint32),
        in_specs=[
            pl.BlockSpec(memory_space=pltpu.VMEM),
            pl.BlockSpec(memory_space=pltpu.VMEM),
            pl.BlockSpec(memory_space=pltpu.VMEM),
            pl.BlockSpec(memory_space=pltpu.SMEM),
        ],
        out_specs=pl.BlockSpec(memory_space=pltpu.VMEM),
    )(conf_map, fc_w, fc_b, lam.reshape(1))

    sc_topk = functools.partial(
        pl.kernel,
        mesh=plsc.VectorSubcoreMesh(core_axis_name="c", subcore_axis_name="s"),
        out_type=jax.ShapeDtypeStruct((N, HW), jnp.float32),
        scratch_types=[
            pltpu.VMEM((HW,), jnp.uint32),
            pltpu.VMEM((HW,), jnp.float32),
            pltpu.VMEM((4 * 16 * 256,), jnp.int32),
            pltpu.VMEM((HW,), jnp.int32),
        ],
        compiler_params=pltpu.CompilerParams(needs_layout_passes=False),
    )(functools.partial(_sc_topk_body, N=N, Vb=Vb, HW=HW, K=K))
    mask = sc_topk(keys.reshape(N, HW)).reshape(N, H, W)

    CB = 32
    xm, x_fuse = pl.pallas_call(
        _apply_body,
        grid=(B, C // CB),
        in_specs=[
            pl.BlockSpec((Vb, CB, H, W), lambda b, c: (b, c, 0, 0)),
            pl.BlockSpec((Vb, H, W), lambda b, c: (b, 0, 0)),
        ],
        out_specs=[
            pl.BlockSpec((Vb, CB, H, W), lambda b, c: (b, c, 0, 0)),
            pl.BlockSpec((1, CB, H, W), lambda b, c: (b, c, 0, 0)),
        ],
        out_shape=[
            jax.ShapeDtypeStruct((N, C, H, W), jnp.float32),
            jax.ShapeDtypeStruct((B, C, H, W), jnp.float32),
        ],
        compiler_params=pltpu.CompilerParams(
            dimension_semantics=("parallel", "parallel"),
        ),
    )(x, mask)

    hw = jnp.float32(HW)
    rates = jnp.float32(Vb * K) / (record_len.astype(jnp.float32) * hw)
    communication_rates = jnp.sum(rates) / jnp.float32(B)

    return (x_fuse, communication_rates, xm, jnp.float32(0.0))


# speedup vs baseline: 1.2359x; 1.0014x over previous
import functools

import jax
import jax.numpy as jnp
from jax import lax
from jax.experimental import pallas as pl
from jax.experimental.pallas import tpu as pltpu
from jax.experimental.pallas import tpu_sc as plsc

_MINT = -2147483648


def _keys_body(conf_ref, fcw_ref, fcb_ref, lam_ref, keys_ref, *, N, B, H, W):
    Vb = N // B
    conf = conf_ref[...]
    cm = jax.nn.sigmoid(conf).max(axis=1)
    lam = lam_ref[0]
    fcw = fcw_ref[...]
    fcb = fcb_ref[...]

    tf = (
        jnp.dot(cm.reshape(N * H, W), fcw.T, preferred_element_type=jnp.float32)
        + fcb[None, :]
    ).reshape(N, H, H)

    comms = []
    for v in range(N):
        ego = (v // Vb) * Vb
        req = 1.0 - cm[ego]
        diff = cm[v] * req
        sim = jnp.dot(tf[v], cm[v], preferred_element_type=jnp.float32)
        comms.append(lam * diff + (1.0 - lam) * sim)
    comm = jnp.stack(comms, axis=0)

    ki = lax.bitcast_convert_type(comm, jnp.int32)
    k2 = ki ^ ((ki >> 31) | jnp.int32(_MINT))
    keys_ref[...] = lax.bitcast_convert_type(k2, jnp.uint32)


def _sc_topk_body(keys_hbm, mask_hbm, keys_v, mask_v, hist_v, pref_v,
                  *, N, Vb, HW, K):
    L = 16
    U = 4
    n_chunks = HW // L
    wid = lax.axis_index("s") * 2 + lax.axis_index("c")

    @pl.when(wid < N)
    def _():
        row = wid
        pltpu.sync_copy(keys_hbm.at[row], keys_v)
        lane = lax.broadcasted_iota(jnp.int32, (L,), 0)
        ones = jnp.ones((L,), jnp.int32)

        def one_pass(shift, carry):
            prefix, k_rem = carry
            wmax = jnp.uint32((1 << (shift + 8)) - 1)

            @plsc.parallel_loop(0, U * 256, unroll=16)
            def _(i):
                hist_v[pl.ds(i * L, L)] = jnp.zeros((L,), jnp.int32)

            @plsc.parallel_loop(0, n_chunks, unroll=U)
            def _(i):
                k = keys_v[pl.ds(i * L, L)]
                digit = ((k >> jnp.uint32(shift))
                         & jnp.uint32(255)).astype(jnp.int32)
                addr = (i & (U - 1)) * (256 * L) + lane * 256 + digit
                if shift == 24:
                    plsc.addupdate_scatter(hist_v, [addr], ones)
                else:
                    match = (k - prefix) <= wmax
                    plsc.addupdate_scatter(hist_v, [addr], ones, mask=match)

            def scan_step(j2, c):
                seen, bin_acc, above_acc, size_acc = c
                j = 15 - j2
                tv = jnp.zeros((L,), jnp.int32)
                for u in range(U):
                    for l in range(L):
                        tv = tv + hist_v[pl.ds(u * 256 * L + l * 256 + j * L, L)]
                rv = lax.rev(tv, (0,))
                incl = seen + plsc.cumsum(rv)
                excl = incl - rv
                hit = (excl < k_rem) & (incl >= k_rem)
                bins_desc = j * L + 15 - lane
                bin_acc = bin_acc + jnp.sum(jnp.where(hit, bins_desc, 0))
                above_acc = above_acc + jnp.sum(jnp.where(hit, excl, 0))
                size_acc = size_acc + jnp.sum(jnp.where(hit, rv, 0))
                return (seen + jnp.sum(tv), bin_acc, above_acc, size_acc)

            _, d, above, dsize = lax.fori_loop(
                0, 16, scan_step,
                (jnp.int32(0), jnp.int32(0), jnp.int32(0), jnp.int32(0)))

            prefix = prefix | (d.astype(jnp.uint32) << jnp.uint32(shift))
            return (prefix, k_rem - above), dsize

        carry = (jnp.uint32(0), jnp.int32(K))
        for shift in (24, 16, 8, 0):
            carry, eq_total = one_pass(shift, carry)
        t_key, need = carry

        is_ego = (row % Vb) == 0

        @pl.when(eq_total == need)
        def _():
            @plsc.parallel_loop(0, n_chunks, unroll=8)
            def _(i):
                k = keys_v[pl.ds(i * L, L)]
                sel = (k >= t_key) | is_ego
                mask_v[pl.ds(i * L, L)] = jnp.where(sel, 1.0, 0.0)

        @pl.when(eq_total != need)
        def _():
            def count_step(i, run):
                k = keys_v[pl.ds(i * L, L)]
                eq = k == t_key
                pref_v[pl.ds(i * L, L)] = run
                return run + plsc.all_reduce_population_count(eq)

            lax.fori_loop(0, n_chunks, count_step, jnp.zeros((L,), jnp.int32))

            @plsc.parallel_loop(0, n_chunks, unroll=2)
            def _(i):
                k = keys_v[pl.ds(i * L, L)]
                eq = k == t_key
                cs = pref_v[pl.ds(i * L, L)] + plsc.cumsum(eq.astype(jnp.int32))
                sel = (k > t_key) | (eq & (cs <= need)) | is_ego
                mask_v[pl.ds(i * L, L)] = jnp.where(sel, 1.0, 0.0)

        pltpu.sync_copy(mask_v, mask_hbm.at[row])


def _apply_body(x_ref, mask_ref, xm_ref, fuse_ref):
    xm = x_ref[...] * mask_ref[...][:, None, :, :]
    xm_ref[...] = xm
    fuse_ref[...] = jnp.max(xm, axis=0, keepdims=True)


@jax.jit
def kernel(x, record_len, conf_map, lam, fc_w, fc_b):
    N, C, H, W = x.shape
    B = record_len.shape[0]
    Vb = N // B
    HW = H * W
    K = HW // 2

    keys = pl.pallas_call(
        functools.partial(_keys_body, N=N, B=B, H=H, W=W),
        out_shape=jax.ShapeDtypeStruct((N, H, W), jnp.uint32),
        in_specs=[
            pl.BlockSpec(memory_space=pltpu.VMEM),
            pl.BlockSpec(memory_space=pltpu.VMEM),
            pl.BlockSpec(memory_space=pltpu.VMEM),
            pl.BlockSpec(memory_space=pltpu.SMEM),
        ],
        out_specs=pl.BlockSpec(memory_space=pltpu.VMEM),
    )(conf_map, fc_w, fc_b, lam.reshape(1))

    sc_topk = functools.partial(
        pl.kernel,
        mesh=plsc.VectorSubcoreMesh(core_axis_name="c", subcore_axis_name="s"),
        out_type=jax.ShapeDtypeStruct((N, HW), jnp.float32),
        scratch_types=[
            pltpu.VMEM((HW,), jnp.uint32),
            pltpu.VMEM((HW,), jnp.float32),
            pltpu.VMEM((4 * 16 * 256,), jnp.int32),
            pltpu.VMEM((HW,), jnp.int32),
        ],
        compiler_params=pltpu.CompilerParams(needs_layout_passes=False),
    )(functools.partial(_sc_topk_body, N=N, Vb=Vb, HW=HW, K=K))
    mask = sc_topk(keys.reshape(N, HW)).reshape(N, H, W)

    CB = 32
    xm, x_fuse = pl.pallas_call(
        _apply_body,
        grid=(B, C // CB),
        in_specs=[
            pl.BlockSpec((Vb, CB, H, W), lambda b, c: (b, c, 0, 0)),
            pl.BlockSpec((Vb, H, W), lambda b, c: (b, 0, 0)),
        ],
        out_specs=[
            pl.BlockSpec((Vb, CB, H, W), lambda b, c: (b, c, 0, 0)),
            pl.BlockSpec((1, CB, H, W), lambda b, c: (b, c, 0, 0)),
        ],
        out_shape=[
            jax.ShapeDtypeStruct((N, C, H, W), jnp.float32),
            jax.ShapeDtypeStruct((B, C, H, W), jnp.float32),
        ],
        compiler_params=pltpu.CompilerParams(
            dimension_semantics=("parallel", "parallel"),
        ),
    )(x, mask)

    hw = jnp.float32(HW)
    rates = jnp.float32(Vb * K) / (record_len.astype(jnp.float32) * hw)
    communication_rates = jnp.sum(rates) / jnp.float32(B)

    return (x_fuse, communication_rates, xm, jnp.float32(0.0))
